# baseline (speedup 1.0000x reference)
import functools

import jax
import jax.numpy as jnp
from jax import lax
from jax.experimental import pallas as pl
from jax.experimental.pallas import tpu as pltpu
from jax.experimental.pallas import tpu_sc as plsc

N = 10000
E = 320000
NUM_RES = 25
D = 128
G = 64

NPAD = 10240
STRIPE = NPAD // 16
K = 80
NCHUNK = E // K
CPT = NCHUNK // 32
KD = 80
CPTD = (E // KD) // 32
HW = 32
NH = NPAD * HW
HSTRIPE = NH // 16
F32 = jnp.float32
HI = jax.lax.Precision.HIGHEST

_mesh = plsc.VectorSubcoreMesh(core_axis_name="c", subcore_axis_name="s")


@functools.partial(
    pl.kernel,
    out_type=jax.ShapeDtypeStruct((2, NPAD), F32),
    mesh=_mesh,
    scratch_types=[
        pltpu.VMEM((CPTD, KD), jnp.int32),
        pltpu.VMEM((KD,), F32),
        pltpu.VMEM((STRIPE,), F32),
        pltpu.VMEM_SHARED((NPAD,), F32),
    ],
)
def _sc_degree(eidx, out, dst2d, ones_v, zbuf, deg_sh):
    c = lax.axis_index("c")
    s = lax.axis_index("s")

    def fill_ones(i, _):
        ones_v[pl.ds(i * 16, 16)] = jnp.full((16,), 1.0, F32)
        return 0
    lax.fori_loop(0, KD // 16, fill_ones, 0)

    def fill_z(i, _):
        zbuf[pl.ds(i * 16, 16)] = jnp.zeros((16,), F32)
        return 0
    lax.fori_loop(0, STRIPE // 16, fill_z, 0)

    pltpu.sync_copy(zbuf, deg_sh.at[pl.ds(s * STRIPE, STRIPE)])
    plsc.subcore_barrier()

    w = c * 16 + s
    pltpu.sync_copy(eidx.at[1, w], dst2d)

    def body(i, _):
        pltpu.sync_copy(ones_v, deg_sh.at[dst2d.at[i]], add=True)
        return 0
    lax.fori_loop(0, CPTD, body, 0)

    plsc.subcore_barrier()
    pltpu.sync_copy(deg_sh.at[pl.ds(s * STRIPE, STRIPE)],
                    out.at[c, pl.ds(s * STRIPE, STRIPE)])


@functools.partial(
    pl.kernel,
    out_type=jax.ShapeDtypeStruct((2, NH), F32),
    mesh=_mesh,
    scratch_types=[
        pltpu.VMEM((CPT, K), jnp.int32),
        pltpu.VMEM((K,), jnp.int32),
        pltpu.VMEM((K,), jnp.int32),
        pltpu.VMEM((K,), jnp.int32),
        pltpu.VMEM((K,), F32),
        pltpu.VMEM((K,), jnp.int32),
        pltpu.VMEM((STRIPE,), F32),
        pltpu.VMEM_SHARED((NH,), F32),
    ],
)
def _sc_hist(epk, xpad, dis1d, out, packed, sa, da, xv, dv, qv,
             zbuf, hist_sh):
    c = lax.axis_index("c")
    s = lax.axis_index("s")
    w = c * 16 + s

    def fill_z(i, _):
        zbuf[pl.ds(i * 16, 16)] = jnp.zeros((16,), F32)
        return 0
    lax.fori_loop(0, STRIPE // 16, fill_z, 0)

    def zero_hist(j, _):
        pltpu.sync_copy(zbuf, hist_sh.at[pl.ds(s * HSTRIPE + j * STRIPE,
                                               STRIPE)])
        return 0
    lax.fori_loop(0, HW, zero_hist, 0)

    pltpu.sync_copy(epk.at[w], packed)
    plsc.subcore_barrier()

    def body(i, _):
        def u(q, _):
            sl = pl.ds(q * 16, 16)
            v = packed[i, sl]
            sa[sl] = jnp.bitwise_and(v, jnp.int32(0xFFFF))
            da[sl] = lax.shift_right_logical(v, jnp.int32(16))
            return 0
        lax.fori_loop(0, K // 16, u, 0)
        pltpu.sync_copy(xpad.at[sa], xv)
        pltpu.sync_copy(dis1d.at[sa], dv)

        def mkq(q, _):
            sl = pl.ds(q * 16, 16)
            qv[sl] = lax.shift_left(da[sl], jnp.int32(5)) + xv[sl]
            return 0
        lax.fori_loop(0, K // 16, mkq, 0)
        pltpu.sync_copy(dv, hist_sh.at[qv], add=True)
        return 0
    lax.fori_loop(0, CPT, body, 0)

    plsc.subcore_barrier()
    pltpu.sync_copy(hist_sh.at[pl.ds(s * HSTRIPE, HSTRIPE)],
                    out.at[c, pl.ds(s * HSTRIPE, HSTRIPE)])


@functools.partial(
    pl.kernel,
    out_type=jax.ShapeDtypeStruct((2, NPAD, D), F32),
    mesh=_mesh,
    scratch_types=[
        pltpu.VMEM((CPT, K), jnp.int32),
        pltpu.VMEM((K,), jnp.int32),
        pltpu.VMEM((K,), jnp.int32),
        pltpu.VMEM((K,), jnp.int32),
        pltpu.VMEM((K,), jnp.int32),
        pltpu.VMEM((K, D), F32),
        pltpu.VMEM((K, D), F32),
        pltpu.VMEM_SHARED((NPAD, D), F32),
        pltpu.SemaphoreType.DMA,
        pltpu.SemaphoreType.DMA,
    ],
)
def _sc_aggregate(epk, g, out, packed, sa, da, sb, db, rows_a, rows_b,
                  agg_sh, sem_a, sem_b):
    c = lax.axis_index("c")
    s = lax.axis_index("s")

    def fill_z(i, _):
        rows_a[i // 8, pl.ds((i % 8) * 16, 16)] = jnp.zeros((16,), F32)
        return 0
    lax.fori_loop(0, K * (D // 16), fill_z, 0)

    def zero_stripe(j, _):
        pltpu.sync_copy(rows_a, agg_sh.at[pl.ds(s * STRIPE + j * K, K), :])
        return 0
    lax.fori_loop(0, STRIPE // K, zero_stripe, 0)
    plsc.subcore_barrier()

    w = c * 16 + s
    pltpu.sync_copy(epk.at[w], packed)

    def unpack(i, sbuf, dbuf):
        def u(q, _):
            v = packed[i, pl.ds(q * 16, 16)]
            sbuf[pl.ds(q * 16, 16)] = jnp.bitwise_and(v, jnp.int32(0xFFFF))
            dbuf[pl.ds(q * 16, 16)] = lax.shift_right_logical(v, jnp.int32(16))
            return 0
        lax.fori_loop(0, K // 16, u, 0)

    unpack(0, sa, da)
    pltpu.async_copy(g.at[sa], rows_a, sem_a)

    def pair(j, _):
        i1 = 2 * j + 1
        i2 = 2 * j + 2
        unpack(i1, sb, db)
        pltpu.async_copy(g.at[sb], rows_b, sem_b)
        pltpu.make_async_copy(g.at[sa], rows_a, sem_a).wait()
        pltpu.sync_copy(rows_a, agg_sh.at[da], add=True)
        unpack(i2, sa, da)
        pltpu.async_copy(g.at[sa], rows_a, sem_a)
        pltpu.make_async_copy(g.at[sb], rows_b, sem_b).wait()
        pltpu.sync_copy(rows_b, agg_sh.at[db], add=True)
        return 0
    lax.fori_loop(0, CPT // 2, pair, 0)
    pltpu.make_async_copy(g.at[sa], rows_a, sem_a).wait()
    pltpu.sync_copy(rows_a, agg_sh.at[da], add=True)

    plsc.subcore_barrier()
    pltpu.sync_copy(agg_sh.at[pl.ds(s * STRIPE, STRIPE), :],
                    out.at[c, pl.ds(s * STRIPE, STRIPE), :])


def _stage0_body(deg_ref, dis_ref):
    deg = deg_ref[...]
    dis_ref[...] = jax.lax.rsqrt(deg[0:1, :] + deg[1:2, :] + 1.0)


def _stage2_body(h_ref, deg_ref, x_ref, emb_ref, w1_ref, b1_ref, w2_ref,
                 g2_ref):
    deg = deg_ref[...]
    degsum = deg[:, 0:1] + deg[:, 1:2] + 1.0
    dis = jax.lax.rsqrt(degsum)
    onehot = (x_ref[...] == lax.broadcasted_iota(jnp.int32, (N, NUM_RES), 1)
              ).astype(F32)
    m25 = (h_ref[0] + h_ref[1])[:, :NUM_RES] + dis * onehot
    t1 = jnp.dot(emb_ref[...], w1_ref[...], precision=HI,
                 preferred_element_type=F32)
    h1 = jnp.maximum(dis * jnp.dot(m25, t1, precision=HI,
                                   preferred_element_type=F32)
                     + b1_ref[...], 0.0)
    g2_ref[...] = dis * jnp.dot(h1, w2_ref[...], precision=HI,
                                preferred_element_type=F32)


def _stage3_body(agg_ref, g2_ref, deg_ref, b2_ref, lw_ref, lb_ref, batch_ref,
                 out_ref):
    agg = agg_ref[0] + agg_ref[1]
    deg = deg_ref[...]
    dis = jax.lax.rsqrt(deg[:, 0:1] + deg[:, 1:2] + 1.0)
    h2 = jnp.maximum(dis * (agg + g2_ref[...]) + b2_ref[...], 0.0)
    s = jnp.dot(h2, lw_ref[...], precision=HI, preferred_element_type=F32)
    onehot = (batch_ref[...] == lax.broadcasted_iota(jnp.int32, (N, G), 1)
              ).astype(F32)
    sums = lax.dot_general(s, onehot, (((0,), (0,)), ((), ())), precision=HI,
                           preferred_element_type=F32)
    counts = jnp.sum(onehot, axis=0, keepdims=True)
    out_ref[...] = sums / jnp.maximum(counts, 1.0) + lb_ref[0, 0]


def kernel(x, edge_index, batch, emb, W1, b1, W2, b2, lin_W, lin_b):
    ei32 = edge_index.astype(jnp.int32)
    epk = jnp.bitwise_or(ei32[0], jnp.left_shift(ei32[1], 16)
                         ).reshape(32, CPT, K)
    eidx_d = ei32.reshape(2, 32, CPTD, KD)
    x2 = x.astype(jnp.int32).reshape(N, 1)
    batch2 = batch.astype(jnp.int32).reshape(N, 1)

    deg = _sc_degree(eidx_d)
    degT = jnp.transpose(deg[:, :N])
    xpad = jnp.pad(x.astype(jnp.int32), (0, NPAD - N))

    dis1d = pl.pallas_call(
        _stage0_body,
        out_shape=jax.ShapeDtypeStruct((1, NPAD), F32),
    )(deg).reshape(NPAD)

    hist = _sc_hist(epk, xpad, dis1d)
    hist3 = hist.reshape(2, NPAD, HW)[:, :N, :]

    g2 = pl.pallas_call(
        _stage2_body,
        out_shape=jax.ShapeDtypeStruct((N, D), F32),
    )(hist3, degT, x2, emb, W1, b1.reshape(1, D), W2)

    agg2 = _sc_aggregate(epk, g2)

    out = pl.pallas_call(
        _stage3_body,
        out_shape=jax.ShapeDtypeStruct((1, G), F32),
    )(agg2[:, :N, :], g2, degT, b2.reshape(1, D), lin_W,
      lin_b.reshape(1, 1), batch2)
    return out.reshape(G)


# speedup vs baseline: 1.4888x; 1.4888x over previous
import functools

import jax
import jax.numpy as jnp
from jax import lax
from jax.experimental import pallas as pl
from jax.experimental.pallas import tpu as pltpu
from jax.experimental.pallas import tpu_sc as plsc

N = 10000
E = 320000
NUM_RES = 25
D = 128
G = 64

NPAD = 10240
STRIPE = NPAD // 16
K = 80
NCHUNK = E // K
CPT = NCHUNK // 32
KD = 80
CPTD = (E // KD) // 32
HW = 32
NH = NPAD * HW
HSTRIPE = NH // 16
F32 = jnp.float32
HI = jax.lax.Precision.HIGHEST

_mesh = plsc.VectorSubcoreMesh(core_axis_name="c", subcore_axis_name="s")


@functools.partial(
    pl.kernel,
    out_type=jax.ShapeDtypeStruct((2, NPAD), F32),
    mesh=_mesh,
    scratch_types=[
        pltpu.VMEM((CPTD, KD), jnp.int32),
        pltpu.VMEM((KD,), F32),
        pltpu.VMEM((STRIPE,), F32),
        pltpu.VMEM_SHARED((NPAD,), F32),
    ],
)
def _sc_degree(eidx, out, dst2d, ones_v, zbuf, deg_sh):
    c = lax.axis_index("c")
    s = lax.axis_index("s")

    def fill_ones(i, _):
        ones_v[pl.ds(i * 16, 16)] = jnp.full((16,), 1.0, F32)
        return 0
    lax.fori_loop(0, KD // 16, fill_ones, 0)

    def fill_z(i, _):
        zbuf[pl.ds(i * 16, 16)] = jnp.zeros((16,), F32)
        return 0
    lax.fori_loop(0, STRIPE // 16, fill_z, 0)

    pltpu.sync_copy(zbuf, deg_sh.at[pl.ds(s * STRIPE, STRIPE)])
    plsc.subcore_barrier()

    w = c * 16 + s
    pltpu.sync_copy(eidx.at[1, w], dst2d)

    def body(i, _):
        pltpu.sync_copy(ones_v, deg_sh.at[dst2d.at[i]], add=True)
        return 0
    lax.fori_loop(0, CPTD, body, 0)

    plsc.subcore_barrier()
    pltpu.sync_copy(deg_sh.at[pl.ds(s * STRIPE, STRIPE)],
                    out.at[c, pl.ds(s * STRIPE, STRIPE)])


@functools.partial(
    pl.kernel,
    out_type=jax.ShapeDtypeStruct((2, NH), F32),
    mesh=_mesh,
    scratch_types=[
        pltpu.VMEM((CPT, K), jnp.int32),
        pltpu.VMEM((K,), jnp.int32),
        pltpu.VMEM((K,), jnp.int32),
        pltpu.VMEM((K,), jnp.int32),
        pltpu.VMEM((K,), F32),
        pltpu.VMEM((K,), jnp.int32),
        pltpu.VMEM((STRIPE,), F32),
        pltpu.VMEM((STRIPE,), jnp.int32),
        pltpu.VMEM_SHARED((NH,), F32),
        pltpu.VMEM_SHARED((NPAD,), F32),
        pltpu.VMEM_SHARED((NPAD,), jnp.int32),
    ],
)
def _sc_hist(epk, xpad, dis, out, packed, sa, da, xv, dv, qv,
             fbuf, xibuf, hist_sh, dis_sh, x_sh):
    c = lax.axis_index("c")
    s = lax.axis_index("s")
    w = c * 16 + s

    def fill_z(i, _):
        fbuf[pl.ds(i * 16, 16)] = jnp.zeros((16,), F32)
        return 0
    lax.fori_loop(0, STRIPE // 16, fill_z, 0)

    def zero_hist(j, _):
        pltpu.sync_copy(fbuf, hist_sh.at[pl.ds(s * HSTRIPE + j * STRIPE,
                                               STRIPE)])
        return 0
    lax.fori_loop(0, HW, zero_hist, 0)

    pltpu.sync_copy(dis.at[pl.ds(s * STRIPE, STRIPE)], fbuf)
    pltpu.sync_copy(fbuf, dis_sh.at[pl.ds(s * STRIPE, STRIPE)])
    pltpu.sync_copy(xpad.at[pl.ds(s * STRIPE, STRIPE)], xibuf)
    pltpu.sync_copy(xibuf, x_sh.at[pl.ds(s * STRIPE, STRIPE)])

    pltpu.sync_copy(epk.at[w], packed)
    plsc.subcore_barrier()

    def body(i, _):
        def u(q, _):
            sl = pl.ds(q * 16, 16)
            v = packed[i, sl]
            sa[sl] = jnp.bitwise_and(v, jnp.int32(0xFFFF))
            da[sl] = lax.shift_right_logical(v, jnp.int32(16))
            return 0
        lax.fori_loop(0, K // 16, u, 0)
        pltpu.sync_copy(x_sh.at[sa], xv)
        pltpu.sync_copy(dis_sh.at[sa], dv)

        def mkq(q, _):
            sl = pl.ds(q * 16, 16)
            qv[sl] = lax.shift_left(da[sl], jnp.int32(5)) + xv[sl]
            return 0
        lax.fori_loop(0, K // 16, mkq, 0)
        pltpu.sync_copy(dv, hist_sh.at[qv], add=True)
        return 0
    lax.fori_loop(0, CPT, body, 0)

    plsc.subcore_barrier()
    pltpu.sync_copy(hist_sh.at[pl.ds(s * HSTRIPE, HSTRIPE)],
                    out.at[c, pl.ds(s * HSTRIPE, HSTRIPE)])


@functools.partial(
    pl.kernel,
    out_type=jax.ShapeDtypeStruct((2, NPAD, D), F32),
    mesh=_mesh,
    scratch_types=[
        pltpu.VMEM((CPT, K), jnp.int32),
        pltpu.VMEM((K,), jnp.int32),
        pltpu.VMEM((K,), jnp.int32),
        pltpu.VMEM((K,), jnp.int32),
        pltpu.VMEM((K,), jnp.int32),
        pltpu.VMEM((K, D), F32),
        pltpu.VMEM((K, D), F32),
        pltpu.VMEM_SHARED((NPAD, D), F32),
        pltpu.SemaphoreType.DMA,
        pltpu.SemaphoreType.DMA,
    ],
)
def _sc_aggregate(epk, g, out, packed, sa, da, sb, db, rows_a, rows_b,
                  agg_sh, sem_a, sem_b):
    c = lax.axis_index("c")
    s = lax.axis_index("s")

    def fill_z(i, _):
        rows_a[i // 8, pl.ds((i % 8) * 16, 16)] = jnp.zeros((16,), F32)
        return 0
    lax.fori_loop(0, K * (D // 16), fill_z, 0)

    def zero_stripe(j, _):
        pltpu.sync_copy(rows_a, agg_sh.at[pl.ds(s * STRIPE + j * K, K), :])
        return 0
    lax.fori_loop(0, STRIPE // K, zero_stripe, 0)
    plsc.subcore_barrier()

    w = c * 16 + s
    pltpu.sync_copy(epk.at[w], packed)

    def unpack(i, sbuf, dbuf):
        def u(q, _):
            v = packed[i, pl.ds(q * 16, 16)]
            sbuf[pl.ds(q * 16, 16)] = jnp.bitwise_and(v, jnp.int32(0xFFFF))
            dbuf[pl.ds(q * 16, 16)] = lax.shift_right_logical(v, jnp.int32(16))
            return 0
        lax.fori_loop(0, K // 16, u, 0)

    unpack(0, sa, da)
    pltpu.async_copy(g.at[sa], rows_a, sem_a)

    def pair(j, _):
        i1 = 2 * j + 1
        i2 = 2 * j + 2
        unpack(i1, sb, db)
        pltpu.async_copy(g.at[sb], rows_b, sem_b)
        pltpu.make_async_copy(g.at[sa], rows_a, sem_a).wait()
        pltpu.sync_copy(rows_a, agg_sh.at[da], add=True)
        unpack(i2, sa, da)
        pltpu.async_copy(g.at[sa], rows_a, sem_a)
        pltpu.make_async_copy(g.at[sb], rows_b, sem_b).wait()
        pltpu.sync_copy(rows_b, agg_sh.at[db], add=True)
        return 0
    lax.fori_loop(0, CPT // 2, pair, 0)
    pltpu.make_async_copy(g.at[sa], rows_a, sem_a).wait()
    pltpu.sync_copy(rows_a, agg_sh.at[da], add=True)

    plsc.subcore_barrier()
    pltpu.sync_copy(agg_sh.at[pl.ds(s * STRIPE, STRIPE), :],
                    out.at[c, pl.ds(s * STRIPE, STRIPE), :])


def _stage0_body(deg_ref, dis_ref):
    deg = deg_ref[...]
    dis_ref[...] = jax.lax.rsqrt(deg[0:1, :] + deg[1:2, :] + 1.0)


def _stage2_body(h_ref, deg_ref, x_ref, emb_ref, w1_ref, b1_ref, w2_ref,
                 g2_ref):
    deg = deg_ref[...]
    degsum = deg[:, 0:1] + deg[:, 1:2] + 1.0
    dis = jax.lax.rsqrt(degsum)
    onehot = (x_ref[...] == lax.broadcasted_iota(jnp.int32, (N, NUM_RES), 1)
              ).astype(F32)
    m25 = (h_ref[0] + h_ref[1])[:, :NUM_RES] + dis * onehot
    t1 = jnp.dot(emb_ref[...], w1_ref[...], precision=HI,
                 preferred_element_type=F32)
    h1 = jnp.maximum(dis * jnp.dot(m25, t1, precision=HI,
                                   preferred_element_type=F32)
                     + b1_ref[...], 0.0)
    g2_ref[...] = dis * jnp.dot(h1, w2_ref[...], precision=HI,
                                preferred_element_type=F32)


def _stage3_body(agg_ref, g2_ref, deg_ref, b2_ref, lw_ref, lb_ref, batch_ref,
                 out_ref):
    agg = agg_ref[0] + agg_ref[1]
    deg = deg_ref[...]
    dis = jax.lax.rsqrt(deg[:, 0:1] + deg[:, 1:2] + 1.0)
    h2 = jnp.maximum(dis * (agg + g2_ref[...]) + b2_ref[...], 0.0)
    s = jnp.dot(h2, lw_ref[...], precision=HI, preferred_element_type=F32)
    onehot = (batch_ref[...] == lax.broadcasted_iota(jnp.int32, (N, G), 1)
              ).astype(F32)
    sums = lax.dot_general(s, onehot, (((0,), (0,)), ((), ())), precision=HI,
                           preferred_element_type=F32)
    counts = jnp.sum(onehot, axis=0, keepdims=True)
    out_ref[...] = sums / jnp.maximum(counts, 1.0) + lb_ref[0, 0]


def kernel(x, edge_index, batch, emb, W1, b1, W2, b2, lin_W, lin_b):
    ei32 = edge_index.astype(jnp.int32)
    epk = jnp.bitwise_or(ei32[0], jnp.left_shift(ei32[1], 16)
                         ).reshape(32, CPT, K)
    eidx_d = ei32.reshape(2, 32, CPTD, KD)
    x2 = x.astype(jnp.int32).reshape(N, 1)
    batch2 = batch.astype(jnp.int32).reshape(N, 1)

    deg = _sc_degree(eidx_d)
    degT = jnp.transpose(deg[:, :N])
    xpad = jnp.pad(x.astype(jnp.int32), (0, NPAD - N))

    dis1d = pl.pallas_call(
        _stage0_body,
        out_shape=jax.ShapeDtypeStruct((1, NPAD), F32),
    )(deg).reshape(NPAD)

    hist = _sc_hist(epk, xpad, dis1d)
    hist3 = hist.reshape(2, NPAD, HW)[:, :N, :]

    g2 = pl.pallas_call(
        _stage2_body,
        out_shape=jax.ShapeDtypeStruct((N, D), F32),
    )(hist3, degT, x2, emb, W1, b1.reshape(1, D), W2)

    agg2 = _sc_aggregate(epk, g2)

    out = pl.pallas_call(
        _stage3_body,
        out_shape=jax.ShapeDtypeStruct((1, G), F32),
    )(agg2[:, :N, :], g2, degT, b2.reshape(1, D), lin_W,
      lin_b.reshape(1, 1), batch2)
    return out.reshape(G)


# speedup vs baseline: 1.5800x; 1.0613x over previous
import functools

import jax
import jax.numpy as jnp
from jax import lax
from jax.experimental import pallas as pl
from jax.experimental.pallas import tpu as pltpu
from jax.experimental.pallas import tpu_sc as plsc

N = 10000
E = 320000
NUM_RES = 25
D = 128
G = 64

NPAD = 10240
STRIPE = NPAD // 16
K = 80
NCHUNK = E // K
CPT = NCHUNK // 32
KD = 80
CPTD = (E // KD) // 32
HW = 32
NH = NPAD * HW
HSTRIPE = NH // 16
F32 = jnp.float32
HI = jax.lax.Precision.HIGHEST

_mesh = plsc.VectorSubcoreMesh(core_axis_name="c", subcore_axis_name="s")


@functools.partial(
    pl.kernel,
    out_type=jax.ShapeDtypeStruct((2, NPAD), F32),
    mesh=_mesh,
    scratch_types=[
        pltpu.VMEM((CPTD, KD), jnp.int32),
        pltpu.VMEM((KD,), F32),
        pltpu.VMEM((STRIPE,), F32),
        pltpu.VMEM_SHARED((NPAD,), F32),
    ],
)
def _sc_degree(eidx, out, dst2d, ones_v, zbuf, deg_sh):
    c = lax.axis_index("c")
    s = lax.axis_index("s")

    def fill_ones(i, _):
        ones_v[pl.ds(i * 16, 16)] = jnp.full((16,), 1.0, F32)
        return 0
    lax.fori_loop(0, KD // 16, fill_ones, 0)

    def fill_z(i, _):
        zbuf[pl.ds(i * 16, 16)] = jnp.zeros((16,), F32)
        return 0
    lax.fori_loop(0, STRIPE // 16, fill_z, 0)

    pltpu.sync_copy(zbuf, deg_sh.at[pl.ds(s * STRIPE, STRIPE)])
    plsc.subcore_barrier()

    w = c * 16 + s
    pltpu.sync_copy(eidx.at[1, w], dst2d)

    def body(i, _):
        pltpu.sync_copy(ones_v, deg_sh.at[dst2d.at[i]], add=True)
        return 0
    lax.fori_loop(0, CPTD, body, 0)

    plsc.subcore_barrier()
    pltpu.sync_copy(deg_sh.at[pl.ds(s * STRIPE, STRIPE)],
                    out.at[c, pl.ds(s * STRIPE, STRIPE)])


@functools.partial(
    pl.kernel,
    out_type=jax.ShapeDtypeStruct((2, NH), F32),
    mesh=_mesh,
    scratch_types=[
        pltpu.VMEM((CPT, K), jnp.int32),
        pltpu.VMEM((K,), jnp.int32),
        pltpu.VMEM((K,), jnp.int32),
        pltpu.VMEM((K,), jnp.int32),
        pltpu.VMEM((K,), F32),
        pltpu.VMEM((K,), jnp.int32),
        pltpu.VMEM((STRIPE,), F32),
        pltpu.VMEM((STRIPE,), jnp.int32),
        pltpu.VMEM_SHARED((NH,), F32),
        pltpu.VMEM_SHARED((NPAD,), jnp.int32),
    ],
)
def _sc_hist(epk, pkd, out, packed, sa, da, wv, dv, qv,
             fbuf, xibuf, hist_sh, pk_sh):
    c = lax.axis_index("c")
    s = lax.axis_index("s")
    w = c * 16 + s

    def fill_z(i, _):
        fbuf[pl.ds(i * 16, 16)] = jnp.zeros((16,), F32)
        return 0
    lax.fori_loop(0, STRIPE // 16, fill_z, 0)

    def zero_hist(j, _):
        pltpu.sync_copy(fbuf, hist_sh.at[pl.ds(s * HSTRIPE + j * STRIPE,
                                               STRIPE)])
        return 0
    lax.fori_loop(0, HW, zero_hist, 0)

    pltpu.sync_copy(pkd.at[pl.ds(s * STRIPE, STRIPE)], xibuf)
    pltpu.sync_copy(xibuf, pk_sh.at[pl.ds(s * STRIPE, STRIPE)])

    pltpu.sync_copy(epk.at[w], packed)
    plsc.subcore_barrier()

    def body(i, _):
        def u(q, _):
            sl = pl.ds(q * 16, 16)
            v = packed[i, sl]
            sa[sl] = jnp.bitwise_and(v, jnp.int32(0xFFFF))
            da[sl] = lax.shift_right_logical(v, jnp.int32(16))
            return 0
        lax.fori_loop(0, K // 16, u, 0)
        pltpu.sync_copy(pk_sh.at[sa], wv)

        def mkq(q, _):
            sl = pl.ds(q * 16, 16)
            pw = wv[sl]
            qv[sl] = (lax.shift_left(da[sl], jnp.int32(5))
                      + jnp.bitwise_and(pw, jnp.int32(31)))
            dv[sl] = lax.bitcast_convert_type(
                jnp.bitwise_and(pw, jnp.int32(-32)), F32)
            return 0
        lax.fori_loop(0, K // 16, mkq, 0)
        pltpu.sync_copy(dv, hist_sh.at[qv], add=True)
        return 0
    lax.fori_loop(0, CPT, body, 0)

    plsc.subcore_barrier()
    pltpu.sync_copy(hist_sh.at[pl.ds(s * HSTRIPE, HSTRIPE)],
                    out.at[c, pl.ds(s * HSTRIPE, HSTRIPE)])


@functools.partial(
    pl.kernel,
    out_type=jax.ShapeDtypeStruct((2, NPAD, D), F32),
    mesh=_mesh,
    scratch_types=[
        pltpu.VMEM((CPT, K), jnp.int32),
        pltpu.VMEM((K,), jnp.int32),
        pltpu.VMEM((K,), jnp.int32),
        pltpu.VMEM((K,), jnp.int32),
        pltpu.VMEM((K,), jnp.int32),
        pltpu.VMEM((K, D), F32),
        pltpu.VMEM((K, D), F32),
        pltpu.VMEM_SHARED((NPAD, D), F32),
        pltpu.SemaphoreType.DMA,
        pltpu.SemaphoreType.DMA,
    ],
)
def _sc_aggregate(epk, g, out, packed, sa, da, sb, db, rows_a, rows_b,
                  agg_sh, sem_a, sem_b):
    c = lax.axis_index("c")
    s = lax.axis_index("s")

    def fill_z(i, _):
        rows_a[i // 8, pl.ds((i % 8) * 16, 16)] = jnp.zeros((16,), F32)
        return 0
    lax.fori_loop(0, K * (D // 16), fill_z, 0)

    def zero_stripe(j, _):
        pltpu.sync_copy(rows_a, agg_sh.at[pl.ds(s * STRIPE + j * K, K), :])
        return 0
    lax.fori_loop(0, STRIPE // K, zero_stripe, 0)
    plsc.subcore_barrier()

    w = c * 16 + s
    pltpu.sync_copy(epk.at[w], packed)

    def unpack(i, sbuf, dbuf):
        def u(q, _):
            v = packed[i, pl.ds(q * 16, 16)]
            sbuf[pl.ds(q * 16, 16)] = jnp.bitwise_and(v, jnp.int32(0xFFFF))
            dbuf[pl.ds(q * 16, 16)] = lax.shift_right_logical(v, jnp.int32(16))
            return 0
        lax.fori_loop(0, K // 16, u, 0)

    unpack(0, sa, da)
    pltpu.async_copy(g.at[sa], rows_a, sem_a)

    def pair(j, _):
        i1 = 2 * j + 1
        i2 = 2 * j + 2
        unpack(i1, sb, db)
        pltpu.async_copy(g.at[sb], rows_b, sem_b)
        pltpu.make_async_copy(g.at[sa], rows_a, sem_a).wait()
        pltpu.sync_copy(rows_a, agg_sh.at[da], add=True)
        unpack(i2, sa, da)
        pltpu.async_copy(g.at[sa], rows_a, sem_a)
        pltpu.make_async_copy(g.at[sb], rows_b, sem_b).wait()
        pltpu.sync_copy(rows_b, agg_sh.at[db], add=True)
        return 0
    lax.fori_loop(0, CPT // 2, pair, 0)
    pltpu.make_async_copy(g.at[sa], rows_a, sem_a).wait()
    pltpu.sync_copy(rows_a, agg_sh.at[da], add=True)

    plsc.subcore_barrier()
    pltpu.sync_copy(agg_sh.at[pl.ds(s * STRIPE, STRIPE), :],
                    out.at[c, pl.ds(s * STRIPE, STRIPE), :])


def _stage0_body(deg_ref, x_ref, pk_ref):
    deg = deg_ref[...]
    dis = jax.lax.rsqrt(deg[0:1, :] + deg[1:2, :] + 1.0)
    bits = lax.bitcast_convert_type(dis, jnp.int32) + jnp.int32(16)
    pk_ref[...] = jnp.bitwise_or(jnp.bitwise_and(bits, jnp.int32(-32)),
                                 x_ref[...])


def _stage2_body(h_ref, deg_ref, x_ref, emb_ref, w1_ref, b1_ref, w2_ref,
                 g2_ref):
    deg = deg_ref[...]
    degsum = deg[:, 0:1] + deg[:, 1:2] + 1.0
    dis = jax.lax.rsqrt(degsum)
    onehot = (x_ref[...] == lax.broadcasted_iota(jnp.int32, (N, NUM_RES), 1)
              ).astype(F32)
    m25 = (h_ref[0] + h_ref[1])[:, :NUM_RES] + dis * onehot
    t1 = jnp.dot(emb_ref[...], w1_ref[...], precision=HI,
                 preferred_element_type=F32)
    h1 = jnp.maximum(dis * jnp.dot(m25, t1, precision=HI,
                                   preferred_element_type=F32)
                     + b1_ref[...], 0.0)
    g2_ref[...] = dis * jnp.dot(h1, w2_ref[...], precision=HI,
                                preferred_element_type=F32)


def _stage3_body(agg_ref, g2_ref, deg_ref, b2_ref, lw_ref, lb_ref, batch_ref,
                 out_ref):
    agg = agg_ref[0] + agg_ref[1]
    deg = deg_ref[...]
    dis = jax.lax.rsqrt(deg[:, 0:1] + deg[:, 1:2] + 1.0)
    h2 = jnp.maximum(dis * (agg + g2_ref[...]) + b2_ref[...], 0.0)
    s = jnp.dot(h2, lw_ref[...], precision=HI, preferred_element_type=F32)
    onehot = (batch_ref[...] == lax.broadcasted_iota(jnp.int32, (N, G), 1)
              ).astype(F32)
    sums = lax.dot_general(s, onehot, (((0,), (0,)), ((), ())), precision=HI,
                           preferred_element_type=F32)
    counts = jnp.sum(onehot, axis=0, keepdims=True)
    out_ref[...] = sums / jnp.maximum(counts, 1.0) + lb_ref[0, 0]


def kernel(x, edge_index, batch, emb, W1, b1, W2, b2, lin_W, lin_b):
    ei32 = edge_index.astype(jnp.int32)
    epk = jnp.bitwise_or(ei32[0], jnp.left_shift(ei32[1], 16)
                         ).reshape(32, CPT, K)
    eidx_d = ei32.reshape(2, 32, CPTD, KD)
    x2 = x.astype(jnp.int32).reshape(N, 1)
    batch2 = batch.astype(jnp.int32).reshape(N, 1)

    deg = _sc_degree(eidx_d)
    degT = jnp.transpose(deg[:, :N])
    xpad = jnp.pad(x.astype(jnp.int32), (0, NPAD - N))

    pk1d = pl.pallas_call(
        _stage0_body,
        out_shape=jax.ShapeDtypeStruct((1, NPAD), jnp.int32),
    )(deg, xpad.reshape(1, NPAD)).reshape(NPAD)

    hist = _sc_hist(epk, pk1d)
    hist3 = hist.reshape(2, NPAD, HW)[:, :N, :]

    g2 = pl.pallas_call(
        _stage2_body,
        out_shape=jax.ShapeDtypeStruct((N, D), F32),
    )(hist3, degT, x2, emb, W1, b1.reshape(1, D), W2)

    agg2 = _sc_aggregate(epk, g2)

    out = pl.pallas_call(
        _stage3_body,
        out_shape=jax.ShapeDtypeStruct((1, G), F32),
    )(agg2[:, :N, :], g2, degT, b2.reshape(1, D), lin_W,
      lin_b.reshape(1, 1), batch2)
    return out.reshape(G)
